# baseline (speedup 1.0000x reference)
import functools

import jax
import jax.numpy as jnp
from jax import lax
from jax.experimental import pallas as pl
from jax.experimental.pallas import tpu as pltpu
from jax.experimental.pallas import tpu_sc as plsc

_BLK = 2000
_NUM_GRAPHS = 512


def _mm_body(x_ref, w_ref, o_ref):
    o_ref[...] = jnp.dot(x_ref[...], w_ref[...],
                         preferred_element_type=jnp.float32)


def _matmul(x, w):
    n, k = x.shape
    d = w.shape[1]
    return pl.pallas_call(
        _mm_body,
        grid=(n // _BLK,),
        in_specs=[pl.BlockSpec((_BLK, k), lambda i: (i, 0)),
                  pl.BlockSpec((k, d), lambda i: (0, 0))],
        out_specs=pl.BlockSpec((_BLK, d), lambda i: (i, 0)),
        out_shape=jax.ShapeDtypeStruct((n, d), jnp.float32),
    )(x, w)


def _mlp_body(y_ref, a0_ref, a1_ref, b1_ref, w2_ref, b2_ref,
              z_ref, s_ref, q_ref):
    i = pl.program_id(0)
    t = jnp.maximum(y_ref[...] + a0_ref[...] + a1_ref[...] + b1_ref[...], 0.0)
    z = jnp.dot(t, w2_ref[...], preferred_element_type=jnp.float32)
    z = jnp.maximum(z + b2_ref[...], 0.0)
    z_ref[...] = z

    @pl.when(i == 0)
    def _():
        s_ref[...] = jnp.zeros_like(s_ref)
        q_ref[...] = jnp.zeros_like(q_ref)

    s_ref[...] += jnp.sum(z, axis=0, keepdims=True)
    q_ref[...] += jnp.sum(z * z, axis=0, keepdims=True)


def _mlp2(y, a0, a1, b1, w2, b2):
    n, d = y.shape
    row = pl.BlockSpec((_BLK, d), lambda i: (i, 0))
    one = pl.BlockSpec((1, d), lambda i: (0, 0))
    return pl.pallas_call(
        _mlp_body,
        grid=(n // _BLK,),
        in_specs=[row, row, row, one, pl.BlockSpec((d, d), lambda i: (0, 0)),
                  one],
        out_specs=[row, one, one],
        out_shape=[jax.ShapeDtypeStruct((n, d), jnp.float32),
                   jax.ShapeDtypeStruct((1, d), jnp.float32),
                   jax.ShapeDtypeStruct((1, d), jnp.float32)],
    )(y, a0, a1, b1, w2, b2)


def _bn_body(z_ref, s_ref, q_ref, g_ref, b_ref, o_ref, *, n):
    m = s_ref[...] * (1.0 / n)
    v = q_ref[...] * (1.0 / n) - m * m
    o_ref[...] = (z_ref[...] - m) * lax.rsqrt(v + 1e-5) * g_ref[...] + b_ref[...]


def _bnorm(z, s, q, g, b):
    n, d = z.shape
    row = pl.BlockSpec((_BLK, d), lambda i: (i, 0))
    one = pl.BlockSpec((1, d), lambda i: (0, 0))
    return pl.pallas_call(
        functools.partial(_bn_body, n=float(n)),
        grid=(n // _BLK,),
        in_specs=[row, one, one, one, one],
        out_specs=row,
        out_shape=jax.ShapeDtypeStruct((n, d), jnp.float32),
    )(z, s, q, g, b)


def _hist_body(ib_ref, st_ref, ct_ref, *, nblk, g):
    i = pl.program_id(0)
    ib = ib_ref[...]
    io = lax.broadcasted_iota(jnp.int32, (ib.shape[0], g), 1)
    oh = (ib == io).astype(jnp.int32)

    @pl.when(i == 0)
    def _():
        ct_ref[...] = jnp.zeros_like(ct_ref)

    ct_ref[...] += jnp.sum(oh, axis=0, keepdims=True)

    @pl.when(i == nblk - 1)
    def _():
        c = ct_ref[...].astype(jnp.float32)
        r = lax.broadcasted_iota(jnp.int32, (g, g), 0)
        cc = lax.broadcasted_iota(jnp.int32, (g, g), 1)
        lt = (r < cc).astype(jnp.float32)
        st = jnp.dot(c, lt, preferred_element_type=jnp.float32)
        st_ref[...] = st.astype(jnp.int32)


def _hist(ib2, g):
    n = ib2.shape[0]
    blk = 1000
    nblk = n // blk
    return pl.pallas_call(
        functools.partial(_hist_body, nblk=nblk, g=g),
        grid=(nblk,),
        in_specs=[pl.BlockSpec((blk, 1), lambda i: (i, 0))],
        out_specs=[pl.BlockSpec((1, g), lambda i: (0, 0)),
                   pl.BlockSpec((1, g), lambda i: (0, 0))],
        out_shape=[jax.ShapeDtypeStruct((1, g), jnp.int32),
                   jax.ShapeDtypeStruct((1, g), jnp.int32)],
    )(ib2)


def _segmax_body(st_ref, ct_ref, x_ref, o_ref):
    gi = pl.program_id(0)
    neg = jnp.full((8, 128), -jnp.inf, dtype=jnp.float32)
    for r in range(8):
        g = gi * 8 + r
        s = st_ref[0, g]
        c = ct_ref[0, g]

        def body(j, acc, s=s, c=c):
            base = s + j * 8
            rows = x_ref[pl.ds(base, 8), :]
            ridx = base + lax.broadcasted_iota(jnp.int32, (8, 128), 0)
            return jnp.maximum(acc, jnp.where(ridx < s + c, rows, -jnp.inf))

        acc = lax.fori_loop(0, (c + 7) // 8, body, neg)
        o_ref[r:r + 1, :] = jnp.max(acc, axis=0, keepdims=True)


def _segmax(xpad, st, ct, g):
    npad, d = xpad.shape
    return pl.pallas_call(
        _segmax_body,
        grid=(g // 8,),
        in_specs=[pl.BlockSpec(memory_space=pltpu.SMEM),
                  pl.BlockSpec(memory_space=pltpu.SMEM),
                  pl.BlockSpec((npad, d), lambda i: (0, 0))],
        out_specs=pl.BlockSpec((8, d), lambda i: (i, 0)),
        out_shape=jax.ShapeDtypeStruct((g, d), jnp.float32),
    )(st, ct, xpad)


_CHUNK = 12672
_GRP = 64
_EB = 2048


def _segsum_sc(y, src, dst):
    n, d = y.shape
    e = src.shape[0]
    nc, ns = 2, 16
    nw = nc * ns
    ep = e // nw
    assert e % nw == 0 and ep % 8 == 0
    nb = -(-ep // _EB)
    e_need = (nw - 1) * ep + nb * _EB
    e_pad = ((e_need + 7) // 8) * 8
    n_pad = ((n + 127) // 128) * 128
    nch = -(-n_pad // _CHUNK)
    trash = _CHUNK

    src = jnp.pad(src, (0, e_pad - e))
    dst = jnp.pad(dst, (0, e_pad - e))

    mesh = plsc.VectorSubcoreMesh(core_axis_name="c", subcore_axis_name="s")

    @functools.partial(
        pl.kernel,
        out_type=jax.ShapeDtypeStruct((2, n_pad, d), jnp.float32),
        mesh=mesh,
        scratch_types=[
            pltpu.VMEM((_EB,), jnp.int32),
            pltpu.VMEM((_EB,), jnp.int32),
            pltpu.VMEM((_GRP,), jnp.int32),
            pltpu.VMEM((_GRP,), jnp.int32),
            pltpu.VMEM((_GRP,), jnp.int32),
            pltpu.VMEM((_GRP,), jnp.int32),
            pltpu.VMEM((_GRP, 128), jnp.float32),
            pltpu.VMEM((_GRP, 128), jnp.float32),
            pltpu.VMEM_SHARED((_CHUNK + 128, 128), jnp.float32),
            pltpu.SemaphoreType.DMA,
            pltpu.SemaphoreType.DMA,
        ],
    )
    def seg_kernel(y_hbm, src_hbm, dst_hbm, agg,
                   seb, deb, gix0, gix1, lix0, lix1, rowb0, rowb1,
                   chunkb, gsem0, gsem1):
        gixs, lixs = (gix0, gix1), (lix0, lix1)
        rowbs, gsems = (rowb0, rowb1), (gsem0, gsem1)
        rowb = rowb0
        cid = lax.axis_index("c")
        sid = lax.axis_index("s")
        wid = sid * nc + cid
        eoff = wid * ep
        lane = lax.iota(jnp.int32, 16)
        z16f = jnp.zeros((16,), jnp.float32)

        def zero_rowb():
            def zb(k, carry):
                rowb[k // 8, pl.ds((k % 8) * 16, 16)] = z16f
                return carry
            lax.fori_loop(0, _GRP * 8, zb, 0)

        for c in range(nch):
            cbase = c * _CHUNK
            csz = min(_CHUNK, n_pad - cbase)
            assert csz % (ns * 8) == 0
            rows_t = csz // ns
            stripe = sid * rows_t

            zero_rowb()
            for k in range(0, rows_t, _GRP):
                ln = min(_GRP, rows_t - k)
                pltpu.sync_copy(rowb.at[pl.ds(0, ln)],
                                chunkb.at[pl.ds(stripe + k, ln)])
            plsc.subcore_barrier()

            def blk_body(blk, carry):
                pltpu.sync_copy(src_hbm.at[pl.ds(eoff + blk * _EB, _EB)], seb)
                pltpu.sync_copy(dst_hbm.at[pl.ds(eoff + blk * _EB, _EB)], deb)

                ngrp = _EB // _GRP
                prev = None
                for g in range(ngrp):
                    p = g & 1
                    for i in range(_GRP // 16):
                        o16 = g * _GRP + i * 16
                        dv = deb[pl.ds(o16, 16)]
                        sv = seb[pl.ds(o16, 16)]
                        gi = blk * _EB + o16 + lane
                        m = (gi < ep) & (dv >= cbase) & (dv < cbase + csz)
                        gixs[p][pl.ds(i * 16, 16)] = jnp.where(m, sv, 0)
                        lixs[p][pl.ds(i * 16, 16)] = jnp.where(m, dv - cbase,
                                                               trash)
                    dsc = pltpu.async_copy(y_hbm.at[gixs[p]], rowbs[p],
                                           gsems[p])
                    if prev is not None:
                        prev.wait()
                        q = 1 - p
                        pltpu.sync_copy(rowbs[q], chunkb.at[lixs[q]],
                                        add=True)
                    prev = dsc
                prev.wait()
                q = (ngrp - 1) & 1
                pltpu.sync_copy(rowbs[q], chunkb.at[lixs[q]], add=True)
                return carry

            lax.fori_loop(0, nb, blk_body, 0)
            plsc.subcore_barrier()

            for k in range(0, rows_t, _GRP):
                ln = min(_GRP, rows_t - k)
                loc = stripe + k
                pltpu.sync_copy(chunkb.at[pl.ds(loc, ln)],
                                agg.at[cid].at[pl.ds(cbase + loc, ln)])
            plsc.subcore_barrier()

    out = seg_kernel(y, src, dst)
    return out[0], out[1]


def kernel(drug_feature, drug_adj, ibatch, W0a, b0a, W0b, b0b,
           Wa, ba, Wb, bb, gamma, beta):
    n = drug_feature.shape[0]
    d = W0a.shape[1]
    src = drug_adj[0]
    dst = drug_adj[1]
    params = [
        (W0a, b0a, W0b, b0b, gamma[0], beta[0]),
        (Wa[0], ba[0], Wb[0], bb[0], gamma[1], beta[1]),
        (Wa[1], ba[1], Wb[1], bb[1], gamma[2], beta[2]),
    ]

    x = drug_feature
    outs = []
    for (w1, b1, w2, b2, g, b) in params:
        y = _matmul(x, w1)
        a0, a1 = _segsum_sc(y, src, dst)
        z, s1, s2 = _mlp2(y, a0, a1, b1.reshape(1, d), w2, b2.reshape(1, d))
        xn = _bnorm(z, s1, s2, g.reshape(1, d), b.reshape(1, d))
        outs.append(xn)
        x = xn

    st, ct = _hist(ibatch.reshape(n, 1), _NUM_GRAPHS)
    res = [_segmax(jnp.pad(o, ((0, 8), (0, 0))), st, ct, _NUM_GRAPHS)
           for o in outs]
    return jnp.concatenate(res, axis=1)


# speedup vs baseline: 17.5625x; 17.5625x over previous
import functools

import jax
import jax.numpy as jnp
from jax import lax
from jax.experimental import pallas as pl
from jax.experimental.pallas import tpu as pltpu
from jax.experimental.pallas import tpu_sc as plsc

_BLK = 2000
_NUM_GRAPHS = 512


def _mm_body(x_ref, w_ref, o_ref):
    o_ref[...] = jnp.dot(x_ref[...], w_ref[...],
                         preferred_element_type=jnp.float32)


def _matmul(x, w):
    n, k = x.shape
    d = w.shape[1]
    return pl.pallas_call(
        _mm_body,
        grid=(n // _BLK,),
        in_specs=[pl.BlockSpec((_BLK, k), lambda i: (i, 0)),
                  pl.BlockSpec((k, d), lambda i: (0, 0))],
        out_specs=pl.BlockSpec((_BLK, d), lambda i: (i, 0)),
        out_shape=jax.ShapeDtypeStruct((n, d), jnp.float32),
    )(x, w)


def _mlp_body(y_ref, a0_ref, b1_ref, w2_ref, b2_ref,
              z_ref, s_ref, q_ref):
    i = pl.program_id(0)
    t = jnp.maximum(y_ref[...] + a0_ref[...] + b1_ref[...], 0.0)
    z = jnp.dot(t, w2_ref[...], preferred_element_type=jnp.float32)
    z = jnp.maximum(z + b2_ref[...], 0.0)
    z_ref[...] = z

    @pl.when(i == 0)
    def _():
        s_ref[...] = jnp.zeros_like(s_ref)
        q_ref[...] = jnp.zeros_like(q_ref)

    s_ref[...] += jnp.sum(z, axis=0, keepdims=True)
    q_ref[...] += jnp.sum(z * z, axis=0, keepdims=True)


def _mlp2(y, a0, b1, w2, b2):
    n, d = y.shape
    row = pl.BlockSpec((_BLK, d), lambda i: (i, 0))
    one = pl.BlockSpec((1, d), lambda i: (0, 0))
    return pl.pallas_call(
        _mlp_body,
        grid=(n // _BLK,),
        in_specs=[row, row, one, pl.BlockSpec((d, d), lambda i: (0, 0)),
                  one],
        out_specs=[row, one, one],
        out_shape=[jax.ShapeDtypeStruct((n, d), jnp.float32),
                   jax.ShapeDtypeStruct((1, d), jnp.float32),
                   jax.ShapeDtypeStruct((1, d), jnp.float32)],
    )(y, a0, b1, w2, b2)


def _bn_body(z_ref, s_ref, q_ref, g_ref, b_ref, o_ref, *, n):
    m = s_ref[...] * (1.0 / n)
    v = q_ref[...] * (1.0 / n) - m * m
    o_ref[...] = (z_ref[...] - m) * lax.rsqrt(v + 1e-5) * g_ref[...] + b_ref[...]


def _bnorm(z, s, q, g, b):
    n, d = z.shape
    row = pl.BlockSpec((_BLK, d), lambda i: (i, 0))
    one = pl.BlockSpec((1, d), lambda i: (0, 0))
    return pl.pallas_call(
        functools.partial(_bn_body, n=float(n)),
        grid=(n // _BLK,),
        in_specs=[row, one, one, one, one],
        out_specs=row,
        out_shape=jax.ShapeDtypeStruct((n, d), jnp.float32),
    )(z, s, q, g, b)


def _hist_body(ib_ref, st_ref, ct_ref, *, nblk, g):
    i = pl.program_id(0)
    ib = ib_ref[...]
    io = lax.broadcasted_iota(jnp.int32, (ib.shape[0], g), 1)
    oh = (ib == io).astype(jnp.int32)

    @pl.when(i == 0)
    def _():
        ct_ref[...] = jnp.zeros_like(ct_ref)

    ct_ref[...] += jnp.sum(oh, axis=0, keepdims=True)

    @pl.when(i == nblk - 1)
    def _():
        c = ct_ref[...].astype(jnp.float32)
        r = lax.broadcasted_iota(jnp.int32, (g, g), 0)
        cc = lax.broadcasted_iota(jnp.int32, (g, g), 1)
        lt = (r < cc).astype(jnp.float32)
        st = jnp.dot(c, lt, preferred_element_type=jnp.float32)
        st_ref[...] = st.astype(jnp.int32)


def _hist(ib2, g):
    n = ib2.shape[0]
    blk = 1000
    nblk = n // blk
    return pl.pallas_call(
        functools.partial(_hist_body, nblk=nblk, g=g),
        grid=(nblk,),
        in_specs=[pl.BlockSpec((blk, 1), lambda i: (i, 0))],
        out_specs=[pl.BlockSpec((1, g), lambda i: (0, 0)),
                   pl.BlockSpec((1, g), lambda i: (0, 0))],
        out_shape=[jax.ShapeDtypeStruct((1, g), jnp.int32),
                   jax.ShapeDtypeStruct((1, g), jnp.int32)],
    )(ib2)


def _segmax_body(st_ref, ct_ref, x_ref, o_ref):
    gi = pl.program_id(0)
    neg = jnp.full((8, 128), -jnp.inf, dtype=jnp.float32)
    for r in range(8):
        g = gi * 8 + r
        s = st_ref[0, g]
        c = ct_ref[0, g]

        def body(j, acc, s=s, c=c):
            base = s + j * 8
            rows = x_ref[pl.ds(base, 8), :]
            ridx = base + lax.broadcasted_iota(jnp.int32, (8, 128), 0)
            return jnp.maximum(acc, jnp.where(ridx < s + c, rows, -jnp.inf))

        acc = lax.fori_loop(0, (c + 7) // 8, body, neg)
        o_ref[r:r + 1, :] = jnp.max(acc, axis=0, keepdims=True)


def _segmax(xpad, st, ct, g):
    npad, d = xpad.shape
    return pl.pallas_call(
        _segmax_body,
        grid=(g // 8,),
        in_specs=[pl.BlockSpec(memory_space=pltpu.SMEM),
                  pl.BlockSpec(memory_space=pltpu.SMEM),
                  pl.BlockSpec((npad, d), lambda i: (0, 0))],
        out_specs=pl.BlockSpec((8, d), lambda i: (i, 0)),
        out_shape=jax.ShapeDtypeStruct((g, d), jnp.float32),
    )(st, ct, xpad)


_ECH = 2000


def _seg_tc_body(src_ref, dst_ref, y_ref, agg_ref):
    i = pl.program_id(0)

    @pl.when(i == 0)
    def _():
        agg_ref[...] = jnp.zeros_like(agg_ref)

    def body(j, carry):
        s = src_ref[0, 0, j]
        d = dst_ref[0, 0, j]
        agg_ref[pl.ds(d, 1), :] += y_ref[pl.ds(s, 1), :]
        return carry

    lax.fori_loop(0, _ECH, body, 0)


def _segsum_tc(y, src2, dst2):
    n, d = y.shape
    g = src2.shape[0]
    sm = pl.BlockSpec((1, 1, _ECH), lambda i: (i, 0, 0),
                      memory_space=pltpu.SMEM)
    return pl.pallas_call(
        _seg_tc_body,
        grid=(g,),
        in_specs=[sm, sm, pl.BlockSpec((n, d), lambda i: (0, 0))],
        out_specs=pl.BlockSpec((n, d), lambda i: (0, 0)),
        out_shape=jax.ShapeDtypeStruct((n, d), jnp.float32),
    )(src2, dst2, y)


_CHUNK = 12672
_GRP = 64
_EB = 2048


def _segsum_sc(y, src, dst):
    n, d = y.shape
    e = src.shape[0]
    nc, ns = 2, 16
    nw = nc * ns
    ep = e // nw
    assert e % nw == 0 and ep % 8 == 0
    nb = -(-ep // _EB)
    e_need = (nw - 1) * ep + nb * _EB
    e_pad = ((e_need + 7) // 8) * 8
    n_pad = ((n + 127) // 128) * 128
    nch = -(-n_pad // _CHUNK)
    trash = _CHUNK

    src = jnp.pad(src, (0, e_pad - e))
    dst = jnp.pad(dst, (0, e_pad - e))

    mesh = plsc.VectorSubcoreMesh(core_axis_name="c", subcore_axis_name="s")

    @functools.partial(
        pl.kernel,
        out_type=jax.ShapeDtypeStruct((2, n_pad, d), jnp.float32),
        mesh=mesh,
        scratch_types=[
            pltpu.VMEM((_EB,), jnp.int32),
            pltpu.VMEM((_EB,), jnp.int32),
            pltpu.VMEM((_GRP,), jnp.int32),
            pltpu.VMEM((_GRP,), jnp.int32),
            pltpu.VMEM((_GRP,), jnp.int32),
            pltpu.VMEM((_GRP,), jnp.int32),
            pltpu.VMEM((_GRP, 128), jnp.float32),
            pltpu.VMEM((_GRP, 128), jnp.float32),
            pltpu.VMEM_SHARED((_CHUNK + 128, 128), jnp.float32),
            pltpu.SemaphoreType.DMA,
            pltpu.SemaphoreType.DMA,
        ],
    )
    def seg_kernel(y_hbm, src_hbm, dst_hbm, agg,
                   seb, deb, gix0, gix1, lix0, lix1, rowb0, rowb1,
                   chunkb, gsem0, gsem1):
        gixs, lixs = (gix0, gix1), (lix0, lix1)
        rowbs, gsems = (rowb0, rowb1), (gsem0, gsem1)
        rowb = rowb0
        cid = lax.axis_index("c")
        sid = lax.axis_index("s")
        wid = sid * nc + cid
        eoff = wid * ep
        lane = lax.iota(jnp.int32, 16)
        z16f = jnp.zeros((16,), jnp.float32)

        def zero_rowb():
            def zb(k, carry):
                rowb[k // 8, pl.ds((k % 8) * 16, 16)] = z16f
                return carry
            lax.fori_loop(0, _GRP * 8, zb, 0)

        for c in range(nch):
            cbase = c * _CHUNK
            csz = min(_CHUNK, n_pad - cbase)
            assert csz % (ns * 8) == 0
            rows_t = csz // ns
            stripe = sid * rows_t

            zero_rowb()
            for k in range(0, rows_t, _GRP):
                ln = min(_GRP, rows_t - k)
                pltpu.sync_copy(rowb.at[pl.ds(0, ln)],
                                chunkb.at[pl.ds(stripe + k, ln)])
            plsc.subcore_barrier()

            def blk_body(blk, carry):
                pltpu.sync_copy(src_hbm.at[pl.ds(eoff + blk * _EB, _EB)], seb)
                pltpu.sync_copy(dst_hbm.at[pl.ds(eoff + blk * _EB, _EB)], deb)

                ngrp = _EB // _GRP
                prev = None
                for g in range(ngrp):
                    p = g & 1
                    for i in range(_GRP // 16):
                        o16 = g * _GRP + i * 16
                        dv = deb[pl.ds(o16, 16)]
                        sv = seb[pl.ds(o16, 16)]
                        gi = blk * _EB + o16 + lane
                        m = (gi < ep) & (dv >= cbase) & (dv < cbase + csz)
                        gixs[p][pl.ds(i * 16, 16)] = jnp.where(m, sv, 0)
                        lixs[p][pl.ds(i * 16, 16)] = jnp.where(m, dv - cbase,
                                                               trash)
                    dsc = pltpu.async_copy(y_hbm.at[gixs[p]], rowbs[p],
                                           gsems[p])
                    if prev is not None:
                        prev.wait()
                        q = 1 - p
                        pltpu.sync_copy(rowbs[q], chunkb.at[lixs[q]],
                                        add=True)
                    prev = dsc
                prev.wait()
                q = (ngrp - 1) & 1
                pltpu.sync_copy(rowbs[q], chunkb.at[lixs[q]], add=True)
                return carry

            lax.fori_loop(0, nb, blk_body, 0)
            plsc.subcore_barrier()

            for k in range(0, rows_t, _GRP):
                ln = min(_GRP, rows_t - k)
                loc = stripe + k
                pltpu.sync_copy(chunkb.at[pl.ds(loc, ln)],
                                agg.at[cid].at[pl.ds(cbase + loc, ln)])
            plsc.subcore_barrier()

    out = seg_kernel(y, src, dst)
    return out[0], out[1]


def kernel(drug_feature, drug_adj, ibatch, W0a, b0a, W0b, b0b,
           Wa, ba, Wb, bb, gamma, beta):
    n = drug_feature.shape[0]
    d = W0a.shape[1]
    src = drug_adj[0]
    dst = drug_adj[1]
    params = [
        (W0a, b0a, W0b, b0b, gamma[0], beta[0]),
        (Wa[0], ba[0], Wb[0], bb[0], gamma[1], beta[1]),
        (Wa[1], ba[1], Wb[1], bb[1], gamma[2], beta[2]),
    ]

    src2 = src.reshape(-1, 1, _ECH)
    dst2 = dst.reshape(-1, 1, _ECH)
    x = drug_feature
    outs = []
    for (w1, b1, w2, b2, g, b) in params:
        y = _matmul(x, w1)
        a = _segsum_tc(y, src2, dst2)
        z, s1, s2 = _mlp2(y, a, b1.reshape(1, d), w2, b2.reshape(1, d))
        xn = _bnorm(z, s1, s2, g.reshape(1, d), b.reshape(1, d))
        outs.append(xn)
        x = xn

    st, ct = _hist(ibatch.reshape(n, 1), _NUM_GRAPHS)
    res = [_segmax(jnp.pad(o, ((0, 8), (0, 0))), st, ct, _NUM_GRAPHS)
           for o in outs]
    return jnp.concatenate(res, axis=1)


# speedup vs baseline: 22.5903x; 1.2863x over previous
import functools

import jax
import jax.numpy as jnp
from jax import lax
from jax.experimental import pallas as pl
from jax.experimental.pallas import tpu as pltpu
from jax.experimental.pallas import tpu_sc as plsc

_BLK = 2000
_NUM_GRAPHS = 512


def _mm_body(x_ref, w_ref, o_ref):
    o_ref[...] = jnp.dot(x_ref[...], w_ref[...],
                         preferred_element_type=jnp.float32)


def _matmul(x, w):
    n, k = x.shape
    d = w.shape[1]
    return pl.pallas_call(
        _mm_body,
        grid=(n // _BLK,),
        in_specs=[pl.BlockSpec((_BLK, k), lambda i: (i, 0)),
                  pl.BlockSpec((k, d), lambda i: (0, 0))],
        out_specs=pl.BlockSpec((_BLK, d), lambda i: (i, 0)),
        out_shape=jax.ShapeDtypeStruct((n, d), jnp.float32),
    )(x, w)


def _mlp_body(y_ref, a0_ref, b1_ref, w2_ref, b2_ref,
              z_ref, s_ref, q_ref):
    i = pl.program_id(0)
    t = jnp.maximum(y_ref[...] + a0_ref[...] + b1_ref[...], 0.0)
    z = jnp.dot(t, w2_ref[...], preferred_element_type=jnp.float32)
    z = jnp.maximum(z + b2_ref[...], 0.0)
    z_ref[...] = z

    @pl.when(i == 0)
    def _():
        s_ref[...] = jnp.zeros_like(s_ref)
        q_ref[...] = jnp.zeros_like(q_ref)

    s_ref[...] += jnp.sum(z, axis=0, keepdims=True)
    q_ref[...] += jnp.sum(z * z, axis=0, keepdims=True)


def _mlp2(y, a0, b1, w2, b2):
    n, d = y.shape
    row = pl.BlockSpec((_BLK, d), lambda i: (i, 0))
    one = pl.BlockSpec((1, d), lambda i: (0, 0))
    return pl.pallas_call(
        _mlp_body,
        grid=(n // _BLK,),
        in_specs=[row, row, one, pl.BlockSpec((d, d), lambda i: (0, 0)),
                  one],
        out_specs=[row, one, one],
        out_shape=[jax.ShapeDtypeStruct((n, d), jnp.float32),
                   jax.ShapeDtypeStruct((1, d), jnp.float32),
                   jax.ShapeDtypeStruct((1, d), jnp.float32)],
    )(y, a0, b1, w2, b2)


def _bn_body(z_ref, s_ref, q_ref, g_ref, b_ref, o_ref, *, n):
    m = s_ref[...] * (1.0 / n)
    v = q_ref[...] * (1.0 / n) - m * m
    o_ref[...] = (z_ref[...] - m) * lax.rsqrt(v + 1e-5) * g_ref[...] + b_ref[...]


def _bnorm(z, s, q, g, b):
    n, d = z.shape
    row = pl.BlockSpec((_BLK, d), lambda i: (i, 0))
    one = pl.BlockSpec((1, d), lambda i: (0, 0))
    return pl.pallas_call(
        functools.partial(_bn_body, n=float(n)),
        grid=(n // _BLK,),
        in_specs=[row, one, one, one, one],
        out_specs=row,
        out_shape=jax.ShapeDtypeStruct((n, d), jnp.float32),
    )(z, s, q, g, b)


def _hist_body(ib_ref, st_ref, ct_ref, *, nblk, g):
    i = pl.program_id(0)
    ib = ib_ref[...]
    io = lax.broadcasted_iota(jnp.int32, (ib.shape[0], g), 1)
    oh = (ib == io).astype(jnp.int32)

    @pl.when(i == 0)
    def _():
        ct_ref[...] = jnp.zeros_like(ct_ref)

    ct_ref[...] += jnp.sum(oh, axis=0, keepdims=True)

    @pl.when(i == nblk - 1)
    def _():
        c = ct_ref[...].astype(jnp.float32)
        r = lax.broadcasted_iota(jnp.int32, (g, g), 0)
        cc = lax.broadcasted_iota(jnp.int32, (g, g), 1)
        lt = (r < cc).astype(jnp.float32)
        st = jnp.dot(c, lt, preferred_element_type=jnp.float32)
        st_ref[...] = st.astype(jnp.int32)


def _hist(ib2, g):
    n = ib2.shape[0]
    blk = 1000
    nblk = n // blk
    return pl.pallas_call(
        functools.partial(_hist_body, nblk=nblk, g=g),
        grid=(nblk,),
        in_specs=[pl.BlockSpec((blk, 1), lambda i: (i, 0))],
        out_specs=[pl.BlockSpec((1, g), lambda i: (0, 0)),
                   pl.BlockSpec((1, g), lambda i: (0, 0))],
        out_shape=[jax.ShapeDtypeStruct((1, g), jnp.int32),
                   jax.ShapeDtypeStruct((1, g), jnp.int32)],
    )(ib2)


def _segmax_body(st_ref, ct_ref, x_ref, o_ref):
    gi = pl.program_id(0)
    neg = jnp.full((8, 128), -jnp.inf, dtype=jnp.float32)
    for r in range(8):
        g = gi * 8 + r
        s = st_ref[0, g]
        c = ct_ref[0, g]

        def body(j, acc, s=s, c=c):
            base = s + j * 8
            rows = x_ref[pl.ds(base, 8), :]
            ridx = base + lax.broadcasted_iota(jnp.int32, (8, 128), 0)
            return jnp.maximum(acc, jnp.where(ridx < s + c, rows, -jnp.inf))

        acc = lax.fori_loop(0, (c + 7) // 8, body, neg)
        o_ref[r:r + 1, :] = jnp.max(acc, axis=0, keepdims=True)


def _segmax(xpad, st, ct, g):
    npad, d = xpad.shape
    return pl.pallas_call(
        _segmax_body,
        grid=(g // 8,),
        in_specs=[pl.BlockSpec(memory_space=pltpu.SMEM),
                  pl.BlockSpec(memory_space=pltpu.SMEM),
                  pl.BlockSpec((npad, d), lambda i: (0, 0))],
        out_specs=pl.BlockSpec((8, d), lambda i: (i, 0)),
        out_shape=jax.ShapeDtypeStruct((g, d), jnp.float32),
    )(st, ct, xpad)


_ECH = 8000


def _seg_tc_body(src_ref, dst_ref, y_ref, agg_ref):
    i = pl.program_id(0)

    @pl.when(i == 0)
    def _():
        agg_ref[...] = jnp.zeros_like(agg_ref)

    def body(j, carry):
        s1 = src_ref[0, 0, 2 * j]
        d1 = dst_ref[0, 0, 2 * j]
        s2 = src_ref[0, 0, 2 * j + 1]
        d2 = dst_ref[0, 0, 2 * j + 1]
        r1 = y_ref[pl.ds(s1, 1), :]
        r2 = y_ref[pl.ds(s2, 1), :]
        agg_ref[pl.ds(d1, 1), :] += r1
        agg_ref[pl.ds(d2, 1), :] += r2
        return carry

    lax.fori_loop(0, _ECH // 2, body, 0)


def _segsum_tc(y, src2, dst2):
    n, d = y.shape
    g = src2.shape[0]
    sm = pl.BlockSpec((1, 1, _ECH), lambda i: (i, 0, 0),
                      memory_space=pltpu.SMEM)
    return pl.pallas_call(
        _seg_tc_body,
        grid=(g,),
        in_specs=[sm, sm, pl.BlockSpec((n, d), lambda i: (0, 0))],
        out_specs=pl.BlockSpec((n, d), lambda i: (0, 0)),
        out_shape=jax.ShapeDtypeStruct((n, d), jnp.float32),
    )(src2, dst2, y)


_CHUNK = 12672
_GRP = 64
_EB = 2048


def _segsum_sc(y, src, dst):
    n, d = y.shape
    e = src.shape[0]
    nc, ns = 2, 16
    nw = nc * ns
    ep = e // nw
    assert e % nw == 0 and ep % 8 == 0
    nb = -(-ep // _EB)
    e_need = (nw - 1) * ep + nb * _EB
    e_pad = ((e_need + 7) // 8) * 8
    n_pad = ((n + 127) // 128) * 128
    nch = -(-n_pad // _CHUNK)
    trash = _CHUNK

    src = jnp.pad(src, (0, e_pad - e))
    dst = jnp.pad(dst, (0, e_pad - e))

    mesh = plsc.VectorSubcoreMesh(core_axis_name="c", subcore_axis_name="s")

    @functools.partial(
        pl.kernel,
        out_type=jax.ShapeDtypeStruct((2, n_pad, d), jnp.float32),
        mesh=mesh,
        scratch_types=[
            pltpu.VMEM((_EB,), jnp.int32),
            pltpu.VMEM((_EB,), jnp.int32),
            pltpu.VMEM((_GRP,), jnp.int32),
            pltpu.VMEM((_GRP,), jnp.int32),
            pltpu.VMEM((_GRP,), jnp.int32),
            pltpu.VMEM((_GRP,), jnp.int32),
            pltpu.VMEM((_GRP, 128), jnp.float32),
            pltpu.VMEM((_GRP, 128), jnp.float32),
            pltpu.VMEM_SHARED((_CHUNK + 128, 128), jnp.float32),
            pltpu.SemaphoreType.DMA,
            pltpu.SemaphoreType.DMA,
        ],
    )
    def seg_kernel(y_hbm, src_hbm, dst_hbm, agg,
                   seb, deb, gix0, gix1, lix0, lix1, rowb0, rowb1,
                   chunkb, gsem0, gsem1):
        gixs, lixs = (gix0, gix1), (lix0, lix1)
        rowbs, gsems = (rowb0, rowb1), (gsem0, gsem1)
        rowb = rowb0
        cid = lax.axis_index("c")
        sid = lax.axis_index("s")
        wid = sid * nc + cid
        eoff = wid * ep
        lane = lax.iota(jnp.int32, 16)
        z16f = jnp.zeros((16,), jnp.float32)

        def zero_rowb():
            def zb(k, carry):
                rowb[k // 8, pl.ds((k % 8) * 16, 16)] = z16f
                return carry
            lax.fori_loop(0, _GRP * 8, zb, 0)

        for c in range(nch):
            cbase = c * _CHUNK
            csz = min(_CHUNK, n_pad - cbase)
            assert csz % (ns * 8) == 0
            rows_t = csz // ns
            stripe = sid * rows_t

            zero_rowb()
            for k in range(0, rows_t, _GRP):
                ln = min(_GRP, rows_t - k)
                pltpu.sync_copy(rowb.at[pl.ds(0, ln)],
                                chunkb.at[pl.ds(stripe + k, ln)])
            plsc.subcore_barrier()

            def blk_body(blk, carry):
                pltpu.sync_copy(src_hbm.at[pl.ds(eoff + blk * _EB, _EB)], seb)
                pltpu.sync_copy(dst_hbm.at[pl.ds(eoff + blk * _EB, _EB)], deb)

                ngrp = _EB // _GRP
                prev = None
                for g in range(ngrp):
                    p = g & 1
                    for i in range(_GRP // 16):
                        o16 = g * _GRP + i * 16
                        dv = deb[pl.ds(o16, 16)]
                        sv = seb[pl.ds(o16, 16)]
                        gi = blk * _EB + o16 + lane
                        m = (gi < ep) & (dv >= cbase) & (dv < cbase + csz)
                        gixs[p][pl.ds(i * 16, 16)] = jnp.where(m, sv, 0)
                        lixs[p][pl.ds(i * 16, 16)] = jnp.where(m, dv - cbase,
                                                               trash)
                    dsc = pltpu.async_copy(y_hbm.at[gixs[p]], rowbs[p],
                                           gsems[p])
                    if prev is not None:
                        prev.wait()
                        q = 1 - p
                        pltpu.sync_copy(rowbs[q], chunkb.at[lixs[q]],
                                        add=True)
                    prev = dsc
                prev.wait()
                q = (ngrp - 1) & 1
                pltpu.sync_copy(rowbs[q], chunkb.at[lixs[q]], add=True)
                return carry

            lax.fori_loop(0, nb, blk_body, 0)
            plsc.subcore_barrier()

            for k in range(0, rows_t, _GRP):
                ln = min(_GRP, rows_t - k)
                loc = stripe + k
                pltpu.sync_copy(chunkb.at[pl.ds(loc, ln)],
                                agg.at[cid].at[pl.ds(cbase + loc, ln)])
            plsc.subcore_barrier()

    out = seg_kernel(y, src, dst)
    return out[0], out[1]


def kernel(drug_feature, drug_adj, ibatch, W0a, b0a, W0b, b0b,
           Wa, ba, Wb, bb, gamma, beta):
    n = drug_feature.shape[0]
    d = W0a.shape[1]
    src = drug_adj[0]
    dst = drug_adj[1]
    params = [
        (W0a, b0a, W0b, b0b, gamma[0], beta[0]),
        (Wa[0], ba[0], Wb[0], bb[0], gamma[1], beta[1]),
        (Wa[1], ba[1], Wb[1], bb[1], gamma[2], beta[2]),
    ]

    src2 = src.reshape(-1, 1, _ECH)
    dst2 = dst.reshape(-1, 1, _ECH)
    x = drug_feature
    outs = []
    for (w1, b1, w2, b2, g, b) in params:
        y = _matmul(x, w1)
        a = _segsum_tc(y, src2, dst2)
        z, s1, s2 = _mlp2(y, a, b1.reshape(1, d), w2, b2.reshape(1, d))
        xn = _bnorm(z, s1, s2, g.reshape(1, d), b.reshape(1, d))
        outs.append(xn)
        x = xn

    st, ct = _hist(ibatch.reshape(n, 1), _NUM_GRAPHS)
    res = [_segmax(jnp.pad(o, ((0, 8), (0, 0))), st, ct, _NUM_GRAPHS)
           for o in outs]
    return jnp.concatenate(res, axis=1)


# speedup vs baseline: 30.5546x; 1.3526x over previous
import functools

import jax
import jax.numpy as jnp
from jax import lax
from jax.experimental import pallas as pl
from jax.experimental.pallas import tpu as pltpu
from jax.experimental.pallas import tpu_sc as plsc

_BLK = 2000
_NUM_GRAPHS = 512


def _mm_body(x_ref, w_ref, o_ref):
    o_ref[...] = jnp.dot(x_ref[...], w_ref[...],
                         preferred_element_type=jnp.float32)


def _matmul(x, w):
    n, k = x.shape
    d = w.shape[1]
    return pl.pallas_call(
        _mm_body,
        grid=(n // _BLK,),
        in_specs=[pl.BlockSpec((_BLK, k), lambda i: (i, 0)),
                  pl.BlockSpec((k, d), lambda i: (0, 0))],
        out_specs=pl.BlockSpec((_BLK, d), lambda i: (i, 0)),
        out_shape=jax.ShapeDtypeStruct((n, d), jnp.float32),
    )(x, w)


def _mlp_body(y_ref, a0_ref, b1_ref, w2_ref, b2_ref,
              z_ref, s_ref, q_ref):
    i = pl.program_id(0)
    t = jnp.maximum(y_ref[...] + a0_ref[...] + b1_ref[...], 0.0)
    z = jnp.dot(t, w2_ref[...], preferred_element_type=jnp.float32)
    z = jnp.maximum(z + b2_ref[...], 0.0)
    z_ref[...] = z

    @pl.when(i == 0)
    def _():
        s_ref[...] = jnp.zeros_like(s_ref)
        q_ref[...] = jnp.zeros_like(q_ref)

    s_ref[...] += jnp.sum(z, axis=0, keepdims=True)
    q_ref[...] += jnp.sum(z * z, axis=0, keepdims=True)


def _mlp2(y, a0, b1, w2, b2):
    n, d = y.shape
    row = pl.BlockSpec((_BLK, d), lambda i: (i, 0))
    one = pl.BlockSpec((1, d), lambda i: (0, 0))
    return pl.pallas_call(
        _mlp_body,
        grid=(n // _BLK,),
        in_specs=[row, row, one, pl.BlockSpec((d, d), lambda i: (0, 0)),
                  one],
        out_specs=[row, one, one],
        out_shape=[jax.ShapeDtypeStruct((n, d), jnp.float32),
                   jax.ShapeDtypeStruct((1, d), jnp.float32),
                   jax.ShapeDtypeStruct((1, d), jnp.float32)],
    )(y, a0, b1, w2, b2)


def _bn_body(z_ref, s_ref, q_ref, g_ref, b_ref, o_ref, *, n):
    m = s_ref[...] * (1.0 / n)
    v = q_ref[...] * (1.0 / n) - m * m
    o_ref[...] = (z_ref[...] - m) * lax.rsqrt(v + 1e-5) * g_ref[...] + b_ref[...]


def _bnorm(z, s, q, g, b):
    n, d = z.shape
    row = pl.BlockSpec((_BLK, d), lambda i: (i, 0))
    one = pl.BlockSpec((1, d), lambda i: (0, 0))
    return pl.pallas_call(
        functools.partial(_bn_body, n=float(n)),
        grid=(n // _BLK,),
        in_specs=[row, one, one, one, one],
        out_specs=row,
        out_shape=jax.ShapeDtypeStruct((n, d), jnp.float32),
    )(z, s, q, g, b)


def _hist_body(ib_ref, st_ref, ct_ref, *, nblk, g):
    i = pl.program_id(0)
    ib = ib_ref[...]
    io = lax.broadcasted_iota(jnp.int32, (ib.shape[0], g), 1)
    oh = (ib == io).astype(jnp.int32)

    @pl.when(i == 0)
    def _():
        ct_ref[...] = jnp.zeros_like(ct_ref)

    ct_ref[...] += jnp.sum(oh, axis=0, keepdims=True)

    @pl.when(i == nblk - 1)
    def _():
        c = ct_ref[...].astype(jnp.float32)
        r = lax.broadcasted_iota(jnp.int32, (g, g), 0)
        cc = lax.broadcasted_iota(jnp.int32, (g, g), 1)
        lt = (r < cc).astype(jnp.float32)
        st = jnp.dot(c, lt, preferred_element_type=jnp.float32)
        st_ref[...] = st.astype(jnp.int32)


def _hist(ib2, g):
    n = ib2.shape[0]
    blk = 1000
    nblk = n // blk
    return pl.pallas_call(
        functools.partial(_hist_body, nblk=nblk, g=g),
        grid=(nblk,),
        in_specs=[pl.BlockSpec((blk, 1), lambda i: (i, 0))],
        out_specs=[pl.BlockSpec((1, g), lambda i: (0, 0)),
                   pl.BlockSpec((1, g), lambda i: (0, 0))],
        out_shape=[jax.ShapeDtypeStruct((1, g), jnp.int32),
                   jax.ShapeDtypeStruct((1, g), jnp.int32)],
    )(ib2)


def _segmax_body(st_ref, ct_ref, x_ref, o_ref):
    gi = pl.program_id(0)
    neg = jnp.full((8, 128), -jnp.inf, dtype=jnp.float32)
    for r in range(8):
        g = gi * 8 + r
        s = st_ref[0, g]
        c = ct_ref[0, g]

        def body(j, acc, s=s, c=c):
            base = s + j * 8
            rows = x_ref[pl.ds(base, 8), :]
            ridx = base + lax.broadcasted_iota(jnp.int32, (8, 128), 0)
            return jnp.maximum(acc, jnp.where(ridx < s + c, rows, -jnp.inf))

        acc = lax.fori_loop(0, (c + 7) // 8, body, neg)
        o_ref[r:r + 1, :] = jnp.max(acc, axis=0, keepdims=True)


def _segmax(xpad, st, ct, g):
    npad, d = xpad.shape
    return pl.pallas_call(
        _segmax_body,
        grid=(g // 8,),
        in_specs=[pl.BlockSpec(memory_space=pltpu.SMEM),
                  pl.BlockSpec(memory_space=pltpu.SMEM),
                  pl.BlockSpec((npad, d), lambda i: (0, 0))],
        out_specs=pl.BlockSpec((8, d), lambda i: (i, 0)),
        out_shape=jax.ShapeDtypeStruct((g, d), jnp.float32),
    )(st, ct, xpad)


_ECH = 8000


def _seg_tc_body(src_ref, dst_ref, y_ref, agg_ref):
    i = pl.program_id(0)

    @pl.when(i == 0)
    def _():
        agg_ref[...] = jnp.zeros_like(agg_ref)

    def body(j, carry):
        ss = [src_ref[0, 0, 4 * j + u] for u in range(4)]
        dd = [dst_ref[0, 0, 4 * j + u] for u in range(4)]
        rr = [y_ref[pl.ds(s, 1), :] for s in ss]
        for u in range(4):
            agg_ref[pl.ds(dd[u], 1), :] += rr[u]
        return carry

    lax.fori_loop(0, _ECH // 4, body, 0)


def _segsum_tc(y, src2, dst2):
    n, d = y.shape
    g = src2.shape[0]
    sm = pl.BlockSpec((1, 1, _ECH), lambda i: (i, 0, 0),
                      memory_space=pltpu.SMEM)
    return pl.pallas_call(
        _seg_tc_body,
        grid=(g,),
        in_specs=[sm, sm, pl.BlockSpec((n, d), lambda i: (0, 0))],
        out_specs=pl.BlockSpec((n, d), lambda i: (0, 0)),
        out_shape=jax.ShapeDtypeStruct((n, d), jnp.float32),
    )(src2, dst2, y)


_CHUNK = 12672
_GRP = 64
_EB = 2048


def _segsum_sc(y, src, dst):
    n, d = y.shape
    e = src.shape[0]
    nc, ns = 2, 16
    nw = nc * ns
    ep = e // nw
    assert e % nw == 0 and ep % 8 == 0
    nb = -(-ep // _EB)
    e_need = (nw - 1) * ep + nb * _EB
    e_pad = ((e_need + 7) // 8) * 8
    n_pad = ((n + 127) // 128) * 128
    nch = -(-n_pad // _CHUNK)
    trash = _CHUNK

    src = jnp.pad(src, (0, e_pad - e))
    dst = jnp.pad(dst, (0, e_pad - e))

    mesh = plsc.VectorSubcoreMesh(core_axis_name="c", subcore_axis_name="s")

    @functools.partial(
        pl.kernel,
        out_type=jax.ShapeDtypeStruct((2, n_pad, d), jnp.float32),
        mesh=mesh,
        scratch_types=[
            pltpu.VMEM((_EB,), jnp.int32),
            pltpu.VMEM((_EB,), jnp.int32),
            pltpu.VMEM((_GRP,), jnp.int32),
            pltpu.VMEM((_GRP,), jnp.int32),
            pltpu.VMEM((_GRP,), jnp.int32),
            pltpu.VMEM((_GRP,), jnp.int32),
            pltpu.VMEM((_GRP, 128), jnp.float32),
            pltpu.VMEM((_GRP, 128), jnp.float32),
            pltpu.VMEM_SHARED((_CHUNK + 128, 128), jnp.float32),
            pltpu.SemaphoreType.DMA,
            pltpu.SemaphoreType.DMA,
        ],
    )
    def seg_kernel(y_hbm, src_hbm, dst_hbm, agg,
                   seb, deb, gix0, gix1, lix0, lix1, rowb0, rowb1,
                   chunkb, gsem0, gsem1):
        gixs, lixs = (gix0, gix1), (lix0, lix1)
        rowbs, gsems = (rowb0, rowb1), (gsem0, gsem1)
        rowb = rowb0
        cid = lax.axis_index("c")
        sid = lax.axis_index("s")
        wid = sid * nc + cid
        eoff = wid * ep
        lane = lax.iota(jnp.int32, 16)
        z16f = jnp.zeros((16,), jnp.float32)

        def zero_rowb():
            def zb(k, carry):
                rowb[k // 8, pl.ds((k % 8) * 16, 16)] = z16f
                return carry
            lax.fori_loop(0, _GRP * 8, zb, 0)

        for c in range(nch):
            cbase = c * _CHUNK
            csz = min(_CHUNK, n_pad - cbase)
            assert csz % (ns * 8) == 0
            rows_t = csz // ns
            stripe = sid * rows_t

            zero_rowb()
            for k in range(0, rows_t, _GRP):
                ln = min(_GRP, rows_t - k)
                pltpu.sync_copy(rowb.at[pl.ds(0, ln)],
                                chunkb.at[pl.ds(stripe + k, ln)])
            plsc.subcore_barrier()

            def blk_body(blk, carry):
                pltpu.sync_copy(src_hbm.at[pl.ds(eoff + blk * _EB, _EB)], seb)
                pltpu.sync_copy(dst_hbm.at[pl.ds(eoff + blk * _EB, _EB)], deb)

                ngrp = _EB // _GRP
                prev = None
                for g in range(ngrp):
                    p = g & 1
                    for i in range(_GRP // 16):
                        o16 = g * _GRP + i * 16
                        dv = deb[pl.ds(o16, 16)]
                        sv = seb[pl.ds(o16, 16)]
                        gi = blk * _EB + o16 + lane
                        m = (gi < ep) & (dv >= cbase) & (dv < cbase + csz)
                        gixs[p][pl.ds(i * 16, 16)] = jnp.where(m, sv, 0)
                        lixs[p][pl.ds(i * 16, 16)] = jnp.where(m, dv - cbase,
                                                               trash)
                    dsc = pltpu.async_copy(y_hbm.at[gixs[p]], rowbs[p],
                                           gsems[p])
                    if prev is not None:
                        prev.wait()
                        q = 1 - p
                        pltpu.sync_copy(rowbs[q], chunkb.at[lixs[q]],
                                        add=True)
                    prev = dsc
                prev.wait()
                q = (ngrp - 1) & 1
                pltpu.sync_copy(rowbs[q], chunkb.at[lixs[q]], add=True)
                return carry

            lax.fori_loop(0, nb, blk_body, 0)
            plsc.subcore_barrier()

            for k in range(0, rows_t, _GRP):
                ln = min(_GRP, rows_t - k)
                loc = stripe + k
                pltpu.sync_copy(chunkb.at[pl.ds(loc, ln)],
                                agg.at[cid].at[pl.ds(cbase + loc, ln)])
            plsc.subcore_barrier()

    out = seg_kernel(y, src, dst)
    return out[0], out[1]


def kernel(drug_feature, drug_adj, ibatch, W0a, b0a, W0b, b0b,
           Wa, ba, Wb, bb, gamma, beta):
    n = drug_feature.shape[0]
    d = W0a.shape[1]
    src = drug_adj[0]
    dst = drug_adj[1]
    params = [
        (W0a, b0a, W0b, b0b, gamma[0], beta[0]),
        (Wa[0], ba[0], Wb[0], bb[0], gamma[1], beta[1]),
        (Wa[1], ba[1], Wb[1], bb[1], gamma[2], beta[2]),
    ]

    src2 = src.reshape(-1, 1, _ECH)
    dst2 = dst.reshape(-1, 1, _ECH)
    x = drug_feature
    outs = []
    for (w1, b1, w2, b2, g, b) in params:
        y = _matmul(x, w1)
        a = _segsum_tc(y, src2, dst2)
        z, s1, s2 = _mlp2(y, a, b1.reshape(1, d), w2, b2.reshape(1, d))
        xn = _bnorm(z, s1, s2, g.reshape(1, d), b.reshape(1, d))
        outs.append(xn)
        x = xn

    st, ct = _hist(ibatch.reshape(n, 1), _NUM_GRAPHS)
    res = [_segmax(jnp.pad(o, ((0, 8), (0, 0))), st, ct, _NUM_GRAPHS)
           for o in outs]
    return jnp.concatenate(res, axis=1)
